# baseline (speedup 1.0000x reference)
import functools

import jax
import jax.numpy as jnp
from jax import lax
from jax.experimental import pallas as pl
from jax.experimental.pallas import tpu as pltpu
from jax.experimental.pallas import tpu_sc as plsc

VOCAB = 100000
HID = 128
L = 200
B = 1024
EPS = 1e-12

NW = 32
TOK = B * L
PER_W = TOK // NW
CHUNK = 80
NCHUNK = PER_W // CHUNK
NBUF = 5
NH = HID // 16
COMB_ROWS = 240


def _tree_sum(vs):
    while len(vs) > 1:
        vs = [vs[i] + vs[i + 1] for i in range(0, len(vs) - 1, 2)] + (
            [vs[-1]] if len(vs) % 2 else [])
    return vs[0]


_GATHER_DNUMS = lax.GatherDimensionNumbers(
    offset_dims=(), collapsed_slice_dims=(0,), start_index_map=(0,))


def _shuffle(v, idx):
    return lax.gather(v, idx[:, None], _GATHER_DNUMS, slice_sizes=(1,),
                      mode=lax.GatherScatterMode.PROMISE_IN_BOUNDS)


def _xlane_sum(v):
    lanes = lax.iota(jnp.int32, 16)
    for sh in (8, 4, 2, 1):
        v = v + _shuffle(v, lanes ^ sh)
    return v


def _rsqrt(x):
    i = lax.bitcast_convert_type(x, jnp.int32)
    i = jnp.int32(0x5F3759DF) - (i >> 1)
    y = lax.bitcast_convert_type(i, jnp.float32)
    for _ in range(2):
        y = y * (1.5 - 0.5 * x * y * y)
    return y


def _sc_body(ids_hbm, word_hbm, comb_hbm, gamma_hbm, beta_hbm,
             out_hbm, idx_v, gb_v,
             rows0, rows1, rows2, rows3, rows4,
             sem0, sem1, sem2, sem3, sem4):
    bufs = [rows0, rows1, rows2, rows3, rows4]
    sems = [sem0, sem1, sem2, sem3, sem4]
    c = lax.axis_index("c")
    s = lax.axis_index("s")
    wid = s * 2 + c

    pltpu.sync_copy(ids_hbm.at[pl.ds(wid * PER_W, PER_W)], idx_v)
    pltpu.sync_copy(gamma_hbm, gb_v.at[0])
    pltpu.sync_copy(beta_hbm, gb_v.at[1])

    inv_h = jnp.float32(1.0 / HID)

    def pos0(g):
        return lax.rem(g * CHUNK, L)

    def issue_fill(g, b):
        pltpu.async_copy(comb_hbm.at[pl.ds(pos0(g), CHUNK)], bufs[b],
                         sems[b])

    def wait_fill(b):
        pltpu.make_async_copy(comb_hbm.at[pl.ds(0, CHUNK)], bufs[b],
                              sems[b]).wait()

    def issue_gather(g, b):
        pltpu.async_copy(
            word_hbm.at[idx_v.at[pl.ds(g * CHUNK, CHUNK)]], bufs[b],
            sems[b], add=True)

    def wait_gather(g, b):
        pltpu.make_async_copy(
            word_hbm.at[idx_v.at[pl.ds(g * CHUNK, CHUNK)]], bufs[b],
            sems[b]).wait()

    def issue_wb(g, b):
        pltpu.async_copy(
            bufs[b], out_hbm.at[pl.ds(wid * PER_W + g * CHUNK, CHUNK)],
            sems[b])

    def wait_wb(b):
        pltpu.make_async_copy(bufs[b], out_hbm.at[pl.ds(0, CHUNK)],
                              sems[b]).wait()

    def process(g, b):
        rows_v = bufs[b]

        @plsc.parallel_loop(0, CHUNK, unroll=1)
        def row(i):
            x = [rows_v[i, pl.ds(h * 16, 16)] for h in range(NH)]
            ssum = _xlane_sum(_tree_sum(x)) * inv_h
            qsum = _xlane_sum(_tree_sum([xh * xh for xh in x])) * inv_h
            inv = _rsqrt(qsum - ssum * ssum + EPS)
            for h in range(NH):
                sl = pl.ds(h * 16, 16)
                rows_v[i, sl] = (x[h] - ssum) * inv * gb_v[0, sl] + gb_v[1, sl]

    issue_fill(0, 0)
    wait_fill(0)
    issue_gather(0, 0)
    issue_fill(1, 1)

    def super_chunk(gq, carry):
        for j in range(NBUF):
            g = gq * NBUF + j
            b1 = (j + 1) % NBUF
            b2 = (j + 2) % NBUF

            @pl.when(g + 1 < NCHUNK)
            def _launch_next_gather():
                wait_fill(b1)
                issue_gather(g + 1, b1)

            wait_gather(g, j)
            process(g, j)
            issue_wb(g, j)

            @pl.when(g + 2 < NCHUNK)
            def _prefetch_fill():
                @pl.when(g >= NBUF - 2)
                def _drain_wb():
                    wait_wb(b2)
                issue_fill(g + 2, b2)
        return carry

    lax.fori_loop(0, NCHUNK // NBUF, super_chunk, 0)
    for b in range(NBUF):
        wait_wb(b)


@jax.jit
def _run(ids2, word_emb, comb_ext, ln_gamma, ln_beta):
    mesh = plsc.VectorSubcoreMesh(core_axis_name="c", subcore_axis_name="s")
    k = functools.partial(
        pl.kernel,
        mesh=mesh,
        out_type=jax.ShapeDtypeStruct((TOK, HID), jnp.float32),
        scratch_types=[
            pltpu.VMEM((PER_W,), jnp.int32),
            pltpu.VMEM((2, HID), jnp.float32),
        ] + [pltpu.VMEM((CHUNK, HID), jnp.float32) for _ in range(NBUF)]
          + [pltpu.SemaphoreType.DMA for _ in range(NBUF)],
    )(_sc_body)
    return k(ids2, word_emb, comb_ext, ln_gamma, ln_beta)


def kernel(input_ids, word_emb, pos_emb, type_emb, ln_gamma, ln_beta):
    ids2 = input_ids.reshape(TOK).astype(jnp.int32)
    comb = pos_emb[:L] + type_emb[0][None, :]
    comb_ext = jnp.concatenate([comb, comb[:COMB_ROWS - L]], axis=0)
    out = _run(ids2, word_emb, comb_ext, ln_gamma, ln_beta)
    return out.reshape(B, L, HID)
